# baseline (speedup 1.0000x reference)
import jax
import jax.numpy as jnp
from jax import lax
from jax.experimental import pallas as pl
from jax.experimental.pallas import tpu as pltpu
from jax.experimental.pallas import tpu_sc as plsc

_BATCH = 16384
_ITEM_NUM = 100000
_USER_HALF = 50000
_LAT = 32
_PP = 16
_NC = 2
_NS = 16
_NW = _NC * _NS
_BPW = _BATCH // _NW
_CH = 128
_NCH = _BPW // _CH
_LANES = 16
_CN = 2048
_TSEC = 25600
_G1 = 50


def _repack_body(pb_t, pa_t, pc_t, b0_t, b1_t, a0_t, a1_t, c0_t, c1_t,
                 th_q0, th_q1, th_q2, th_q3,
                 gp_ref, th_ref):
    pad = jnp.zeros((_PP, _CN), jnp.float32)
    m_e = jnp.concatenate(
        [b0_t[...], a0_t[...], c0_t[...], pb_t[...], pa_t[...]], axis=0)
    m_o = jnp.concatenate(
        [b1_t[...], a1_t[...], c1_t[...], pc_t[...], pad], axis=0)

    def rne(x):
        u = lax.bitcast_convert_type(x, jnp.uint32)
        one = jnp.uint32(1)
        half = jnp.uint32(0x7FFF)
        return u + half + jnp.bitwise_and(
            lax.shift_right_logical(u, jnp.uint32(16)), one)

    packed = jnp.bitwise_or(
        lax.shift_right_logical(rne(m_e), jnp.uint32(16)),
        jnp.bitwise_and(rne(m_o), jnp.uint32(0xFFFF0000)))
    gp_ref[...] = jnp.swapaxes(
        lax.bitcast_convert_type(packed, jnp.int32), 0, 1)
    m_t = jnp.concatenate(
        [th_q0[...], th_q1[...], th_q2[...], th_q3[...]], axis=0)
    th_ref[...] = jnp.swapaxes(m_t, 0, 1)


def _repack(prompt_b, prompt_a, prompt_c, b0, b1, a0, a1, c0, c1, theta):
    f32 = jnp.float32
    grid = (_G1,)
    nmax = _ITEM_NUM // _CN
    col = lambda i: (0, jnp.minimum(i, nmax))
    row = lambda i: (i, 0)
    s16 = pl.BlockSpec((_PP, _CN), col)
    s32 = pl.BlockSpec((_LAT, _CN), col)
    tb = _CN // 4
    tsec = _TSEC // tb
    tmax = _ITEM_NUM // tb
    th_specs = [
        pl.BlockSpec((_LAT, tb),
                     lambda i, q=q: (0, jnp.minimum(q * tsec + i, tmax)))
        for q in range(4)
    ]
    return pl.pallas_call(
        _repack_body,
        grid=grid,
        in_specs=[s16, s16, s16, s32, s32, s32, s32, s32, s32] + th_specs,
        out_specs=[pl.BlockSpec((_CN, 128), row),
                   pl.BlockSpec((tb, 128), row)],
        out_shape=[jax.ShapeDtypeStruct((_G1 * _CN, 128), jnp.int32),
                   jax.ShapeDtypeStruct((_TSEC, 128), f32)],
    )(prompt_b.T, prompt_a.T, prompt_c.T, b0.T, b1.T, a0.T, a1.T,
      c0.T, c1.T, theta.T, theta.T, theta.T, theta.T)


def _sc_gather_body(item2_hbm, user_hbm, gp_hbm, th_hbm,
                    op, ot,
                    idx_v, usr_v, mod_v, u4_v,
                    gp_v, th_v,
                    gsem, wsem):
    wid = lax.axis_index("s") * _NC + lax.axis_index("c")
    base = pl.multiple_of(wid * _BPW, _BPW)
    pltpu.sync_copy(item2_hbm.at[pl.ds(base, _BPW)], idx_v)
    pltpu.sync_copy(user_hbm.at[pl.ds(base, _BPW)], usr_v)
    for t in range(_BPW // _LANES):
        sl = pl.ds(t * _LANES, _LANES)
        j = (t * _LANES) // _CH
        csl = pl.ds((t * _LANES) % _CH, _LANES)
        v = idx_v[sl]
        mod_v.at[j][csl] = jnp.where(v >= _ITEM_NUM, v - _ITEM_NUM, v)
        u = usr_v[sl]
        zero = jnp.zeros((_LANES,), jnp.int32)
        tsec = jnp.full((_LANES,), _TSEC, jnp.int32)
        off = (jnp.where(u >= _TSEC, tsec, zero)
               + jnp.where(u >= 2 * _TSEC, tsec, zero)
               + jnp.where(u >= 3 * _TSEC, tsec, zero))
        u4_v.at[j][csl] = u - off
    for j in range(_NCH):
        gathers = [
            pltpu.async_copy(gp_hbm.at[mod_v.at[j]], gp_v, gsem),
            pltpu.async_copy(th_hbm.at[u4_v.at[j]], th_v, gsem),
        ]
        for c in gathers:
            c.wait()
        row = pl.ds(base + j * _CH, _CH)
        writes = [
            pltpu.async_copy(gp_v, op.at[row], wsem),
            pltpu.async_copy(th_v, ot.at[row], wsem),
        ]
        for c in writes:
            c.wait()


def _sc_gather(item2, user, g_packed, th):
    f32 = jnp.float32
    out_type = (jax.ShapeDtypeStruct((_BATCH, 128), jnp.int32),
                jax.ShapeDtypeStruct((_BATCH, 128), f32))
    scratch = [
        pltpu.VMEM((_BPW,), jnp.int32),
        pltpu.VMEM((_BPW,), jnp.int32),
        pltpu.VMEM((_NCH, _CH), jnp.int32),
        pltpu.VMEM((_NCH, _CH), jnp.int32),
        pltpu.VMEM((_CH, 128), jnp.int32),
        pltpu.VMEM((_CH, 128), f32),
        pltpu.SemaphoreType.DMA,
        pltpu.SemaphoreType.DMA,
    ]
    mesh = plsc.VectorSubcoreMesh(core_axis_name="c", subcore_axis_name="s")
    return pl.kernel(
        _sc_gather_body, out_type=out_type, mesh=mesh,
        scratch_types=scratch,
    )(item2, user, g_packed, th)


_RB = 2048


def _combine_body(i2_ref, u_ref, op_ref, ot_ref,
                  sv0, sv1, W1r, fb1r, W2r, fb2r, W3r, fb3r, W4r, fb4r,
                  out_ref):
    f32 = jnp.float32
    w = op_ref[...]
    oe = lax.bitcast_convert_type(lax.shift_left(w, 16), f32)
    oo = lax.bitcast_convert_type(
        jnp.bitwise_and(w, jnp.int32(-65536)), f32)
    ot = ot_ref[...]
    sel = i2_ref[...] >= _ITEM_NUM
    latb = jnp.where(sel, oo[:, 0:32], oe[:, 0:32])
    lata = jnp.where(sel, oo[:, 32:64], oe[:, 32:64])
    latc = jnp.where(sel, oo[:, 64:96], oe[:, 64:96])
    pb = oe[:, 96:112]
    pa = oe[:, 112:128]
    pc = oo[:, 96:112]

    def lin(p, x, W, fb):
        return (jnp.dot(p, W[:_PP], preferred_element_type=f32)
                + jnp.dot(x, W[_PP:], preferred_element_type=f32) + fb)

    zb = lin(pb, latb, W1r[...], fb1r[...])
    za = lin(pa, lata, W2r[...], fb2r[...])
    zc = lin(pc, latc, W3r[...], fb3r[...])

    u = u_ref[...]
    th = jnp.where(u < _TSEC, ot[:, 0:32],
                   jnp.where(u < 2 * _TSEC, ot[:, 32:64],
                             jnp.where(u < 3 * _TSEC, ot[:, 64:96],
                                       ot[:, 96:128])))
    W4 = W4r[...]
    sv = jnp.where(u < _USER_HALF,
                   jnp.dot(sv0[...], W4[:_PP], preferred_element_type=f32),
                   jnp.dot(sv1[...], W4[:_PP], preferred_element_type=f32))
    zt = sv + jnp.dot(th, W4[_PP:], preferred_element_type=f32) + fb4r[...]

    nc = jax.nn.sigmoid(zc)
    nb = 8.0 * (jax.nn.sigmoid(zb) - 0.5)
    na = jax.nn.sigmoid(za)
    nt = 8.0 * (jax.nn.sigmoid(zt) - 0.5)
    out_ref[...] = nc + (1.0 - nc) / (1.0 + jnp.exp(-1.702 * na * (nt - nb)))


def _combine(i2, usr, op, ot, sv0, sv1, W1, fb1, W2, fb2, W3, fb3,
             W4, fb4):
    grid = (_BATCH // _RB,)
    row = lambda i: (i, 0)
    rep = lambda i: (0, 0)
    idx_spec = pl.BlockSpec((_RB, 1), row)
    g_spec = pl.BlockSpec((_RB, 128), row)
    sv_spec = pl.BlockSpec((1, _PP), rep)
    w_spec = pl.BlockSpec((_PP + _LAT, _LAT), rep)
    fb_spec = pl.BlockSpec((1, _LAT), rep)
    return pl.pallas_call(
        _combine_body,
        grid=grid,
        in_specs=[idx_spec, idx_spec, g_spec, g_spec,
                  sv_spec, sv_spec,
                  w_spec, fb_spec, w_spec, fb_spec,
                  w_spec, fb_spec, w_spec, fb_spec],
        out_specs=pl.BlockSpec((_RB, _LAT), row),
        out_shape=jax.ShapeDtypeStruct((_BATCH, _LAT), jnp.float32),
    )(i2, usr, op, ot, sv0, sv1, W1, fb1, W2, fb2, W3, fb3, W4, fb4)


def kernel(user, item, item2, theta, s_vec0, s_vec1, b0, b1, prompt_b,
           a0, a1, prompt_a, c0, c1, prompt_c,
           W1, fb1, W2, fb2, W3, fb3, W4, fb4):
    del item
    i2 = item2.astype(jnp.int32)
    usr = user.astype(jnp.int32)
    g_packed, th = _repack(prompt_b, prompt_a, prompt_c,
                           b0, b1, a0, a1, c0, c1, theta)
    op, ot = _sc_gather(i2, usr, g_packed, th)
    return _combine(
        i2.reshape(_BATCH, 1), usr.reshape(_BATCH, 1), op, ot,
        s_vec0.reshape(1, _PP), s_vec1.reshape(1, _PP),
        W1, fb1.reshape(1, _LAT), W2, fb2.reshape(1, _LAT),
        W3, fb3.reshape(1, _LAT), W4, fb4.reshape(1, _LAT))


# speedup vs baseline: 1.1083x; 1.1083x over previous
import jax
import jax.numpy as jnp
from jax import lax
from jax.experimental import pallas as pl
from jax.experimental.pallas import tpu as pltpu
from jax.experimental.pallas import tpu_sc as plsc

_BATCH = 16384
_ITEM_NUM = 100000
_USER_HALF = 50000
_LAT = 32
_PP = 16
_NC = 2
_NS = 16
_NW = _NC * _NS
_BPW = _BATCH // _NW
_CH = 128
_NCH = _BPW // _CH
_LANES = 16
_CN = 8192
_TSEC = 26624
_G1 = 13


def _repack_body(pb_t, pa_t, pc_t, b0_t, b1_t, a0_t, a1_t, c0_t, c1_t,
                 th_q0, th_q1, th_q2, th_q3,
                 gp_ref, th_ref):
    pad = jnp.zeros((_PP, _CN), jnp.float32)
    m_e = jnp.concatenate(
        [b0_t[...], a0_t[...], c0_t[...], pb_t[...], pa_t[...]], axis=0)
    m_o = jnp.concatenate(
        [b1_t[...], a1_t[...], c1_t[...], pc_t[...], pad], axis=0)

    def rne(x):
        u = lax.bitcast_convert_type(x, jnp.uint32)
        one = jnp.uint32(1)
        half = jnp.uint32(0x7FFF)
        return u + half + jnp.bitwise_and(
            lax.shift_right_logical(u, jnp.uint32(16)), one)

    packed = jnp.bitwise_or(
        lax.shift_right_logical(rne(m_e), jnp.uint32(16)),
        jnp.bitwise_and(rne(m_o), jnp.uint32(0xFFFF0000)))
    gp_ref[...] = jnp.swapaxes(
        lax.bitcast_convert_type(packed, jnp.int32), 0, 1)
    m_t = jnp.concatenate(
        [th_q0[...], th_q1[...], th_q2[...], th_q3[...]], axis=0)
    th_ref[...] = jnp.swapaxes(m_t, 0, 1)


def _repack(prompt_b, prompt_a, prompt_c, b0, b1, a0, a1, c0, c1, theta):
    f32 = jnp.float32
    grid = (_G1,)
    nmax = _ITEM_NUM // _CN
    col = lambda i: (0, jnp.minimum(i, nmax))
    row = lambda i: (i, 0)
    s16 = pl.BlockSpec((_PP, _CN), col)
    s32 = pl.BlockSpec((_LAT, _CN), col)
    tb = _CN // 4
    tsec = _TSEC // tb
    tmax = _ITEM_NUM // tb
    th_specs = [
        pl.BlockSpec((_LAT, tb),
                     lambda i, q=q: (0, jnp.minimum(q * tsec + i, tmax)))
        for q in range(4)
    ]
    return pl.pallas_call(
        _repack_body,
        grid=grid,
        in_specs=[s16, s16, s16, s32, s32, s32, s32, s32, s32] + th_specs,
        out_specs=[pl.BlockSpec((_CN, 128), row),
                   pl.BlockSpec((tb, 128), row)],
        out_shape=[jax.ShapeDtypeStruct((_G1 * _CN, 128), jnp.int32),
                   jax.ShapeDtypeStruct((_TSEC, 128), f32)],
    )(prompt_b.T, prompt_a.T, prompt_c.T, b0.T, b1.T, a0.T, a1.T,
      c0.T, c1.T, theta.T, theta.T, theta.T, theta.T)


def _sc_gather_body(item2_hbm, user_hbm, gp_hbm, th_hbm,
                    op, ot,
                    idx_v, usr_v, mod_v, u4_v,
                    gp_v, th_v,
                    gsem, wsem):
    wid = lax.axis_index("s") * _NC + lax.axis_index("c")
    base = pl.multiple_of(wid * _BPW, _BPW)
    pltpu.sync_copy(item2_hbm.at[pl.ds(base, _BPW)], idx_v)
    pltpu.sync_copy(user_hbm.at[pl.ds(base, _BPW)], usr_v)
    for t in range(_BPW // _LANES):
        sl = pl.ds(t * _LANES, _LANES)
        j = (t * _LANES) // _CH
        csl = pl.ds((t * _LANES) % _CH, _LANES)
        v = idx_v[sl]
        mod_v.at[j][csl] = jnp.where(v >= _ITEM_NUM, v - _ITEM_NUM, v)
        u = usr_v[sl]
        zero = jnp.zeros((_LANES,), jnp.int32)
        tsec = jnp.full((_LANES,), _TSEC, jnp.int32)
        off = (jnp.where(u >= _TSEC, tsec, zero)
               + jnp.where(u >= 2 * _TSEC, tsec, zero)
               + jnp.where(u >= 3 * _TSEC, tsec, zero))
        u4_v.at[j][csl] = u - off
    for j in range(_NCH):
        gathers = [
            pltpu.async_copy(gp_hbm.at[mod_v.at[j]], gp_v, gsem),
            pltpu.async_copy(th_hbm.at[u4_v.at[j]], th_v, gsem),
        ]
        for c in gathers:
            c.wait()
        row = pl.ds(base + j * _CH, _CH)
        writes = [
            pltpu.async_copy(gp_v, op.at[row], wsem),
            pltpu.async_copy(th_v, ot.at[row], wsem),
        ]
        for c in writes:
            c.wait()


def _sc_gather(item2, user, g_packed, th):
    f32 = jnp.float32
    out_type = (jax.ShapeDtypeStruct((_BATCH, 128), jnp.int32),
                jax.ShapeDtypeStruct((_BATCH, 128), f32))
    scratch = [
        pltpu.VMEM((_BPW,), jnp.int32),
        pltpu.VMEM((_BPW,), jnp.int32),
        pltpu.VMEM((_NCH, _CH), jnp.int32),
        pltpu.VMEM((_NCH, _CH), jnp.int32),
        pltpu.VMEM((_CH, 128), jnp.int32),
        pltpu.VMEM((_CH, 128), f32),
        pltpu.SemaphoreType.DMA,
        pltpu.SemaphoreType.DMA,
    ]
    mesh = plsc.VectorSubcoreMesh(core_axis_name="c", subcore_axis_name="s")
    return pl.kernel(
        _sc_gather_body, out_type=out_type, mesh=mesh,
        scratch_types=scratch,
    )(item2, user, g_packed, th)


_RB = 2048


def _combine_body(i2_ref, u_ref, op_ref, ot_ref,
                  sv0, sv1, W1r, fb1r, W2r, fb2r, W3r, fb3r, W4r, fb4r,
                  out_ref):
    f32 = jnp.float32
    w = op_ref[...]
    oe = lax.bitcast_convert_type(lax.shift_left(w, 16), f32)
    oo = lax.bitcast_convert_type(
        jnp.bitwise_and(w, jnp.int32(-65536)), f32)
    ot = ot_ref[...]
    sel = i2_ref[...] >= _ITEM_NUM
    latb = jnp.where(sel, oo[:, 0:32], oe[:, 0:32])
    lata = jnp.where(sel, oo[:, 32:64], oe[:, 32:64])
    latc = jnp.where(sel, oo[:, 64:96], oe[:, 64:96])
    pb = oe[:, 96:112]
    pa = oe[:, 112:128]
    pc = oo[:, 96:112]

    def lin(p, x, W, fb):
        return (jnp.dot(p, W[:_PP], preferred_element_type=f32)
                + jnp.dot(x, W[_PP:], preferred_element_type=f32) + fb)

    zb = lin(pb, latb, W1r[...], fb1r[...])
    za = lin(pa, lata, W2r[...], fb2r[...])
    zc = lin(pc, latc, W3r[...], fb3r[...])

    u = u_ref[...]
    th = jnp.where(u < _TSEC, ot[:, 0:32],
                   jnp.where(u < 2 * _TSEC, ot[:, 32:64],
                             jnp.where(u < 3 * _TSEC, ot[:, 64:96],
                                       ot[:, 96:128])))
    W4 = W4r[...]
    sv = jnp.where(u < _USER_HALF,
                   jnp.dot(sv0[...], W4[:_PP], preferred_element_type=f32),
                   jnp.dot(sv1[...], W4[:_PP], preferred_element_type=f32))
    zt = sv + jnp.dot(th, W4[_PP:], preferred_element_type=f32) + fb4r[...]

    nc = jax.nn.sigmoid(zc)
    nb = 8.0 * (jax.nn.sigmoid(zb) - 0.5)
    na = jax.nn.sigmoid(za)
    nt = 8.0 * (jax.nn.sigmoid(zt) - 0.5)
    out_ref[...] = nc + (1.0 - nc) / (1.0 + jnp.exp(-1.702 * na * (nt - nb)))


def _combine(i2, usr, op, ot, sv0, sv1, W1, fb1, W2, fb2, W3, fb3,
             W4, fb4):
    grid = (_BATCH // _RB,)
    row = lambda i: (i, 0)
    rep = lambda i: (0, 0)
    idx_spec = pl.BlockSpec((_RB, 1), row)
    g_spec = pl.BlockSpec((_RB, 128), row)
    sv_spec = pl.BlockSpec((1, _PP), rep)
    w_spec = pl.BlockSpec((_PP + _LAT, _LAT), rep)
    fb_spec = pl.BlockSpec((1, _LAT), rep)
    return pl.pallas_call(
        _combine_body,
        grid=grid,
        in_specs=[idx_spec, idx_spec, g_spec, g_spec,
                  sv_spec, sv_spec,
                  w_spec, fb_spec, w_spec, fb_spec,
                  w_spec, fb_spec, w_spec, fb_spec],
        out_specs=pl.BlockSpec((_RB, _LAT), row),
        out_shape=jax.ShapeDtypeStruct((_BATCH, _LAT), jnp.float32),
    )(i2, usr, op, ot, sv0, sv1, W1, fb1, W2, fb2, W3, fb3, W4, fb4)


def kernel(user, item, item2, theta, s_vec0, s_vec1, b0, b1, prompt_b,
           a0, a1, prompt_a, c0, c1, prompt_c,
           W1, fb1, W2, fb2, W3, fb3, W4, fb4):
    del item
    i2 = item2.astype(jnp.int32)
    usr = user.astype(jnp.int32)
    g_packed, th = _repack(prompt_b, prompt_a, prompt_c,
                           b0, b1, a0, a1, c0, c1, theta)
    op, ot = _sc_gather(i2, usr, g_packed, th)
    return _combine(
        i2.reshape(_BATCH, 1), usr.reshape(_BATCH, 1), op, ot,
        s_vec0.reshape(1, _PP), s_vec1.reshape(1, _PP),
        W1, fb1.reshape(1, _LAT), W2, fb2.reshape(1, _LAT),
        W3, fb3.reshape(1, _LAT), W4, fb4.reshape(1, _LAT))


# speedup vs baseline: 1.1187x; 1.0094x over previous
import jax
import jax.numpy as jnp
from jax import lax
from jax.experimental import pallas as pl
from jax.experimental.pallas import tpu as pltpu
from jax.experimental.pallas import tpu_sc as plsc

_BATCH = 16384
_ITEM_NUM = 100000
_USER_HALF = 50000
_LAT = 32
_PP = 16
_NC = 2
_NS = 16
_NW = _NC * _NS
_BPW = _BATCH // _NW
_CH = 128
_NCH = _BPW // _CH
_LANES = 16
_CN = 8192
_TSEC = 26624
_G1 = 13


def _repack_body(pb_t, pa_t, pc_t, b0_t, b1_t, a0_t, a1_t, c0_t, c1_t,
                 th_q0, th_q1, th_q2, th_q3,
                 gp_ref, th_ref):
    def pk(lo_ref, hi_ref):
        ul = lax.bitcast_convert_type(lo_ref[...], jnp.uint32)
        uh = lax.bitcast_convert_type(hi_ref[...], jnp.uint32)
        return jnp.bitwise_or(
            lax.shift_right_logical(ul, jnp.uint32(16)),
            jnp.bitwise_and(uh, jnp.uint32(0xFFFF0000)))

    pa_only = lax.shift_right_logical(
        lax.bitcast_convert_type(pa_t[...], jnp.uint32), jnp.uint32(16))
    packed = jnp.concatenate(
        [pk(b0_t, b1_t), pk(a0_t, a1_t), pk(c0_t, c1_t), pk(pb_t, pc_t),
         pa_only], axis=0)
    gp_ref[...] = jnp.swapaxes(
        lax.bitcast_convert_type(packed, jnp.int32), 0, 1)
    m_t = jnp.concatenate(
        [th_q0[...], th_q1[...], th_q2[...], th_q3[...]], axis=0)
    th_ref[...] = jnp.swapaxes(m_t, 0, 1)


def _repack(prompt_b, prompt_a, prompt_c, b0, b1, a0, a1, c0, c1, theta):
    f32 = jnp.float32
    grid = (_G1,)
    nmax = _ITEM_NUM // _CN
    col = lambda i: (0, jnp.minimum(i, nmax))
    row = lambda i: (i, 0)
    s16 = pl.BlockSpec((_PP, _CN), col)
    s32 = pl.BlockSpec((_LAT, _CN), col)
    tb = _CN // 4
    tsec = _TSEC // tb
    tmax = _ITEM_NUM // tb
    th_specs = [
        pl.BlockSpec((_LAT, tb),
                     lambda i, q=q: (0, jnp.minimum(q * tsec + i, tmax)))
        for q in range(4)
    ]
    return pl.pallas_call(
        _repack_body,
        grid=grid,
        in_specs=[s16, s16, s16, s32, s32, s32, s32, s32, s32] + th_specs,
        out_specs=[pl.BlockSpec((_CN, 128), row),
                   pl.BlockSpec((tb, 128), row)],
        out_shape=[jax.ShapeDtypeStruct((_G1 * _CN, 128), jnp.int32),
                   jax.ShapeDtypeStruct((_TSEC, 128), f32)],
    )(prompt_b.T, prompt_a.T, prompt_c.T, b0.T, b1.T, a0.T, a1.T,
      c0.T, c1.T, theta.T, theta.T, theta.T, theta.T)


def _sc_gather_body(item2_hbm, user_hbm, gp_hbm, th_hbm,
                    op, ot,
                    idx_v, usr_v, mod_v, u4_v,
                    gp_v, th_v,
                    gsem, wsem):
    wid = lax.axis_index("s") * _NC + lax.axis_index("c")
    base = pl.multiple_of(wid * _BPW, _BPW)
    pltpu.sync_copy(item2_hbm.at[pl.ds(base, _BPW)], idx_v)
    pltpu.sync_copy(user_hbm.at[pl.ds(base, _BPW)], usr_v)
    for t in range(_BPW // _LANES):
        sl = pl.ds(t * _LANES, _LANES)
        j = (t * _LANES) // _CH
        csl = pl.ds((t * _LANES) % _CH, _LANES)
        v = idx_v[sl]
        mod_v.at[j][csl] = jnp.where(v >= _ITEM_NUM, v - _ITEM_NUM, v)
        u = usr_v[sl]
        zero = jnp.zeros((_LANES,), jnp.int32)
        tsec = jnp.full((_LANES,), _TSEC, jnp.int32)
        off = (jnp.where(u >= _TSEC, tsec, zero)
               + jnp.where(u >= 2 * _TSEC, tsec, zero)
               + jnp.where(u >= 3 * _TSEC, tsec, zero))
        u4_v.at[j][csl] = u - off
    for j in range(_NCH):
        gathers = [
            pltpu.async_copy(gp_hbm.at[mod_v.at[j]], gp_v, gsem),
            pltpu.async_copy(th_hbm.at[u4_v.at[j]], th_v, gsem),
        ]
        for c in gathers:
            c.wait()
        row = pl.ds(base + j * _CH, _CH)
        writes = [
            pltpu.async_copy(gp_v, op.at[row], wsem),
            pltpu.async_copy(th_v, ot.at[row], wsem),
        ]
        for c in writes:
            c.wait()


def _sc_gather(item2, user, g_packed, th):
    f32 = jnp.float32
    out_type = (jax.ShapeDtypeStruct((_BATCH, 128), jnp.int32),
                jax.ShapeDtypeStruct((_BATCH, 128), f32))
    scratch = [
        pltpu.VMEM((_BPW,), jnp.int32),
        pltpu.VMEM((_BPW,), jnp.int32),
        pltpu.VMEM((_NCH, _CH), jnp.int32),
        pltpu.VMEM((_NCH, _CH), jnp.int32),
        pltpu.VMEM((_CH, 128), jnp.int32),
        pltpu.VMEM((_CH, 128), f32),
        pltpu.SemaphoreType.DMA,
        pltpu.SemaphoreType.DMA,
    ]
    mesh = plsc.VectorSubcoreMesh(core_axis_name="c", subcore_axis_name="s")
    return pl.kernel(
        _sc_gather_body, out_type=out_type, mesh=mesh,
        scratch_types=scratch,
    )(item2, user, g_packed, th)


_RB = 2048


def _combine_body(i2_ref, u_ref, op_ref, ot_ref,
                  sv0, sv1, W1r, fb1r, W2r, fb2r, W3r, fb3r, W4r, fb4r,
                  out_ref):
    f32 = jnp.float32
    w = op_ref[...]
    oe = lax.bitcast_convert_type(lax.shift_left(w, 16), f32)
    oo = lax.bitcast_convert_type(
        jnp.bitwise_and(w, jnp.int32(-65536)), f32)
    ot = ot_ref[...]
    sel = i2_ref[...] >= _ITEM_NUM
    latb = jnp.where(sel, oo[:, 0:32], oe[:, 0:32])
    lata = jnp.where(sel, oo[:, 32:64], oe[:, 32:64])
    latc = jnp.where(sel, oo[:, 64:96], oe[:, 64:96])
    pb = oe[:, 96:112]
    pa = oe[:, 112:128]
    pc = oo[:, 96:112]

    def lin(p, x, W, fb):
        return (jnp.dot(p, W[:_PP], preferred_element_type=f32)
                + jnp.dot(x, W[_PP:], preferred_element_type=f32) + fb)

    zb = lin(pb, latb, W1r[...], fb1r[...])
    za = lin(pa, lata, W2r[...], fb2r[...])
    zc = lin(pc, latc, W3r[...], fb3r[...])

    u = u_ref[...]
    th = jnp.where(u < _TSEC, ot[:, 0:32],
                   jnp.where(u < 2 * _TSEC, ot[:, 32:64],
                             jnp.where(u < 3 * _TSEC, ot[:, 64:96],
                                       ot[:, 96:128])))
    W4 = W4r[...]
    sv = jnp.where(u < _USER_HALF,
                   jnp.dot(sv0[...], W4[:_PP], preferred_element_type=f32),
                   jnp.dot(sv1[...], W4[:_PP], preferred_element_type=f32))
    zt = sv + jnp.dot(th, W4[_PP:], preferred_element_type=f32) + fb4r[...]

    nc = jax.nn.sigmoid(zc)
    nb = 8.0 * (jax.nn.sigmoid(zb) - 0.5)
    na = jax.nn.sigmoid(za)
    nt = 8.0 * (jax.nn.sigmoid(zt) - 0.5)
    out_ref[...] = nc + (1.0 - nc) / (1.0 + jnp.exp(-1.702 * na * (nt - nb)))


def _combine(i2, usr, op, ot, sv0, sv1, W1, fb1, W2, fb2, W3, fb3,
             W4, fb4):
    grid = (_BATCH // _RB,)
    row = lambda i: (i, 0)
    rep = lambda i: (0, 0)
    idx_spec = pl.BlockSpec((_RB, 1), row)
    g_spec = pl.BlockSpec((_RB, 128), row)
    sv_spec = pl.BlockSpec((1, _PP), rep)
    w_spec = pl.BlockSpec((_PP + _LAT, _LAT), rep)
    fb_spec = pl.BlockSpec((1, _LAT), rep)
    return pl.pallas_call(
        _combine_body,
        grid=grid,
        in_specs=[idx_spec, idx_spec, g_spec, g_spec,
                  sv_spec, sv_spec,
                  w_spec, fb_spec, w_spec, fb_spec,
                  w_spec, fb_spec, w_spec, fb_spec],
        out_specs=pl.BlockSpec((_RB, _LAT), row),
        out_shape=jax.ShapeDtypeStruct((_BATCH, _LAT), jnp.float32),
    )(i2, usr, op, ot, sv0, sv1, W1, fb1, W2, fb2, W3, fb3, W4, fb4)


def kernel(user, item, item2, theta, s_vec0, s_vec1, b0, b1, prompt_b,
           a0, a1, prompt_a, c0, c1, prompt_c,
           W1, fb1, W2, fb2, W3, fb3, W4, fb4):
    del item
    i2 = item2.astype(jnp.int32)
    usr = user.astype(jnp.int32)
    g_packed, th = _repack(prompt_b, prompt_a, prompt_c,
                           b0, b1, a0, a1, c0, c1, theta)
    op, ot = _sc_gather(i2, usr, g_packed, th)
    return _combine(
        i2.reshape(_BATCH, 1), usr.reshape(_BATCH, 1), op, ot,
        s_vec0.reshape(1, _PP), s_vec1.reshape(1, _PP),
        W1, fb1.reshape(1, _LAT), W2, fb2.reshape(1, _LAT),
        W3, fb3.reshape(1, _LAT), W4, fb4.reshape(1, _LAT))
